# baseline (speedup 1.0000x reference)
import functools

import jax
import jax.numpy as jnp
import numpy as np
from jax import lax
from jax.experimental import pallas as pl
from jax.experimental.pallas import tpu as pltpu
from jax.experimental.pallas import tpu_sc as plsc
from jax.sharding import Mesh, PartitionSpec

_TWO_PI = 6.283185307179586
_INV_TWO_PI = 0.15915494309189535
_INV_PIO2 = 0.6366197723675814
_PIO2_1 = 1.5707855224609375
_PIO2_2 = 1.0804334124e-05
_PIO2_3 = 6.0770999344e-11


def _floor_sc(x):
    xf = x.astype(jnp.int32).astype(jnp.float32)
    return jnp.where(xf > x, xf - 1.0, xf)


def _sincos(p, floor):
    kf = floor(p * _INV_PIO2 + 0.5)
    r = ((p - kf * _PIO2_1) - kf * _PIO2_2) - kf * _PIO2_3
    k = kf.astype(jnp.int32)
    r2 = r * r
    sp = r * (1.0 + r2 * (-1.6666667163e-01 + r2 * (8.3333337680e-03
              + r2 * (-1.9841270114e-04 + r2 * 2.7557314297e-06))))
    cp = 1.0 + r2 * (-0.5 + r2 * (4.1666667908e-02
              + r2 * (-1.3888889225e-03 + r2 * 2.4801587642e-05)))
    q = jnp.bitwise_and(k, 3)
    swap = jnp.bitwise_and(q, 1) == 1
    s1 = jnp.where(swap, cp, sp)
    c1 = jnp.where(swap, sp, cp)
    sneg = jnp.bitwise_and(q, 2) == 2
    cneg = jnp.bitwise_and(q + 1, 2) == 2
    return jnp.where(sneg, -s1, s1), jnp.where(cneg, -c1, c1)


def _grid_tables_sc(word_w, freq_w, theta_w, P):
    D = word_w.shape[1]
    D2 = 2 * D
    pps = 8
    nact = P // pps
    mesh = plsc.VectorSubcoreMesh(core_axis_name="c", subcore_axis_name="s")

    @functools.partial(
        pl.kernel,
        mesh=mesh,
        out_type=[
            jax.ShapeDtypeStruct((P, D2), jnp.float32),
            jax.ShapeDtypeStruct((P, D2), jnp.float32),
        ],
        scratch_types=[
            pltpu.VMEM((2, D), jnp.float32),
            pltpu.VMEM((2, D), jnp.float32),
            pltpu.VMEM((2, D), jnp.float32),
            pltpu.VMEM((pps, D2), jnp.float32),
            pltpu.VMEM((pps, D2), jnp.float32),
        ],
    )
    def _k(w_hbm, f_hbm, t_hbm, gr_hbm, gi_hbm, w_v, f_v, t_v, gr_v, gi_v):
        wid = lax.axis_index("s") * 2 + lax.axis_index("c")

        @pl.when(wid < nact)
        def _():
            pltpu.sync_copy(w_hbm, w_v)
            pltpu.sync_copy(f_hbm, f_v)
            pltpu.sync_copy(t_hbm, t_v)
            p0 = wid * pps
            for g in range(D2 // 16):
                c0 = g * 16
                row, col = c0 // D, c0 % D
                wv = w_v[row, pl.ds(col, 16)]
                fv = f_v[row, pl.ds(col, 16)]
                tv = t_v[row, pl.ds(col, 16)]
                tv = tv - _TWO_PI * _floor_sc(tv * _INV_TWO_PI)
                for pi in range(pps):
                    posf = (p0 + pi + 1).astype(jnp.float32)
                    s, c = _sincos(posf * fv + tv, _floor_sc)
                    gr_v[pi, pl.ds(c0, 16)] = wv * c
                    gi_v[pi, pl.ds(c0, 16)] = wv * s
            pltpu.sync_copy(gr_v, gr_hbm.at[pl.ds(p0, pps), :])
            pltpu.sync_copy(gi_v, gi_hbm.at[pl.ds(p0, pps), :])

    return _k(word_w, freq_w, theta_w)


_CB = 16
_HB = 112


def _stream_body(n0, ghr_ref, ghi_ref, gwr_ref, gwi_ref, x_ref, or_ref, oi_ref):
    bci = pl.program_id(0)
    x = x_ref[...]
    is_y = ((bci // n0) % 2) == 1

    @pl.when(jnp.logical_not(is_y))
    def _():
        or_ref[...] = x + ghr_ref[...]
        oi_ref[...] = jnp.broadcast_to(ghi_ref[...], x.shape)

    @pl.when(is_y)
    def _():
        or_ref[...] = x + gwr_ref[...]
        oi_ref[...] = jnp.broadcast_to(gwi_ref[...], x.shape)


def _planes(x, word_w, freq_w, theta_w):
    B, C, H, W = x.shape
    cb, hb = _CB, _HB
    n0 = (C // 2) // cb
    nc = C // cb
    xr = x.reshape(B * C, H, W)

    gr_t, gi_t = _grid_tables_sc(word_w, freq_w, theta_w, H)
    gr = gr_t.T
    gi = gi_t.T
    ghr = gr.reshape(C, H, 1)
    ghi = gi.reshape(C, H, 1)
    gwr = gr.reshape(C, 1, W)
    gwi = gi.reshape(C, 1, W)

    body = functools.partial(_stream_body, n0)
    h_spec = pl.BlockSpec((cb, hb, 1), lambda i, j: (i % nc, j, 0))
    w_spec = pl.BlockSpec((cb, 1, W), lambda i, j: (i % nc, 0, 0))
    io_spec = pl.BlockSpec((cb, hb, W), lambda i, j: (i, j, 0))
    re, im = pl.pallas_call(
        body,
        grid=(B * C // cb, H // hb),
        in_specs=[h_spec, h_spec, w_spec, w_spec, io_spec],
        out_specs=[io_spec, io_spec],
        out_shape=[
            jax.ShapeDtypeStruct((B * C, H, W), jnp.float32),
            jax.ShapeDtypeStruct((B * C, H, W), jnp.float32),
        ],
    )(ghr, ghi, gwr, gwi, xr)
    return re.reshape(B, C, H, W), im.reshape(B, C, H, W)


def kernel(x, word_w, freq_w, theta_w):
    B = x.shape[0]
    devs = jax.devices()
    nd = next(n for n in range(min(len(devs), B), 0, -1) if B % n == 0)
    if nd <= 1:
        re, im = _planes(x, word_w, freq_w, theta_w)
        return lax.complex(re, im)

    mesh = Mesh(np.array(devs[:nd]), ("b",))
    pb = PartitionSpec("b")
    pr = PartitionSpec()
    sharded = jax.shard_map(
        _planes, mesh=mesh,
        in_specs=(pb, pr, pr, pr),
        out_specs=(pb, pb),
        check_vma=False,
    )
    re, im = sharded(x, word_w, freq_w, theta_w)
    return lax.complex(re, im)
